# baseline (speedup 1.0000x reference)
import functools

import jax
import jax.numpy as jnp
from jax import lax
from jax.experimental import pallas as pl
from jax.experimental.pallas import tpu as pltpu
from jax.experimental.pallas import tpu_sc as plsc

N = 10000
K = 128
E = 320000

NC = 2
NS = 16
KH = K // 2
CHUNK = 128
EPT = E // NS
RING = 4
LOOKAHEAD = 2
NCHUNK = 160
EPT_PAD = NCHUNK * CHUNK
ACC_N = 10240
JUNK = N
ZROWS = ACC_N // NS
OROWS = 624


def _prep_edges(edge_index):
  src = edge_index[0].astype(jnp.int32)
  dst = edge_index[1].astype(jnp.int32)
  pad = NS * EPT_PAD - E
  src = jnp.concatenate([src, jnp.zeros((pad,), jnp.int32)])
  dst = jnp.concatenate([dst, jnp.full((pad,), JUNK, jnp.int32)])
  return (src.reshape(NS, NCHUNK, CHUNK), dst.reshape(NS, NCHUNK, CHUNK))


def _sc_body(xl_hbm, xr_hbm, srcp, dstp, srcn, dstn,
             xpl_out, xpr_out, xnl_out, xnr_out,
             src_v, dst_v, zbuf, r0, r1, r2, r3,
             sg0, sg1, sg2, sg3, ss0, ss1, ss2, ss3, acc):
  rows = (r0, r1, r2, r3)
  sem_g = (sg0, sg1, sg2, sg3)
  sem_s = (ss0, ss1, ss2, ss3)
  cid = lax.axis_index("c")
  sid = lax.axis_index("s")

  def _zrow(i, carry):
    for c in range(KH // 16):
      zbuf[i, pl.ds(c * 16, 16)] = jnp.zeros((16,), jnp.float32)
    return carry
  lax.fori_loop(0, CHUNK, _zrow, 0)

  @pl.when(cid == 0)
  def _():
    pltpu.sync_copy(srcp.at[sid], src_v)
    pltpu.sync_copy(dstp.at[sid], dst_v)

  @pl.when(cid == 1)
  def _():
    pltpu.sync_copy(srcn.at[sid], src_v)
    pltpu.sync_copy(dstn.at[sid], dst_v)

  def _pass(x_hbm, out_pos, out_neg):
    for b in range(ZROWS // CHUNK):
      pltpu.sync_copy(zbuf, acc.at[pl.ds(sid * ZROWS + b * CHUNK, CHUNK)])
    plsc.subcore_barrier()

    for b in range(LOOKAHEAD):
      pltpu.async_copy(x_hbm.at[src_v.at[b]], rows[b], sem_g[b])

    def _grp(jj, carry):
      for b in range(RING):
        t = jj * RING + b
        tg = t + LOOKAHEAD
        bg = (b + LOOKAHEAD) % RING

        @pl.when(tg < NCHUNK)
        def _():
          @pl.when(t >= RING - LOOKAHEAD)
          def _():
            pltpu.make_async_copy(rows[bg], acc.at[dst_v.at[t]],
                                  sem_s[bg]).wait()
          pltpu.async_copy(x_hbm.at[src_v.at[tg]], rows[bg], sem_g[bg])

        pltpu.make_async_copy(x_hbm.at[src_v.at[t]], rows[b],
                              sem_g[b]).wait()
        pltpu.async_copy(rows[b], acc.at[dst_v.at[t]], sem_s[b], add=True)
      return carry
    lax.fori_loop(0, NCHUNK // RING, _grp, 0)

    for b in range(RING):
      pltpu.make_async_copy(rows[b], acc.at[dst_v.at[0]], sem_s[b]).wait()
    plsc.subcore_barrier()

    base = sid * OROWS
    rem_base = NS * OROWS
    rem = N - NS * OROWS

    @pl.when(cid == 0)
    def _():
      pltpu.sync_copy(acc.at[pl.ds(base, OROWS)],
                      out_pos.at[pl.ds(base, OROWS)])
      @pl.when(sid == NS - 1)
      def _():
        pltpu.sync_copy(acc.at[pl.ds(rem_base, rem)],
                        out_pos.at[pl.ds(rem_base, rem)])

    @pl.when(cid == 1)
    def _():
      pltpu.sync_copy(acc.at[pl.ds(base, OROWS)],
                      out_neg.at[pl.ds(base, OROWS)])
      @pl.when(sid == NS - 1)
      def _():
        pltpu.sync_copy(acc.at[pl.ds(rem_base, rem)],
                        out_neg.at[pl.ds(rem_base, rem)])
    plsc.subcore_barrier()

  _pass(xl_hbm, xpl_out, xnl_out)
  _pass(xr_hbm, xpr_out, xnr_out)


def _sc_scatter(xl, xr, srcp, dstp, srcn, dstn):
  mesh = plsc.VectorSubcoreMesh(core_axis_name="c", subcore_axis_name="s")
  half = jax.ShapeDtypeStruct((N, KH), jnp.float32)
  return pl.kernel(
      _sc_body,
      out_type=(half, half, half, half),
      mesh=mesh,
      compiler_params=pltpu.CompilerParams(use_tc_tiling_on_sc=False),
      scratch_types=(
          [pltpu.VMEM((NCHUNK, CHUNK), jnp.int32)] * 2
          + [pltpu.VMEM((CHUNK, KH), jnp.float32)] * (1 + RING)
          + [pltpu.SemaphoreType.DMA] * (2 * RING)
          + [pltpu.VMEM_SHARED((ACC_N, KH), jnp.float32)]
      ),
  )(xl, xr, srcp, dstp, srcn, dstn)


ROWS_BLK = 2000


def _mlp_body(x_ref, xpl_ref, xpr_ref, xnl_ref, xnr_ref,
              w1_ref, b1_ref, w2_ref, b2_ref, out_ref):
  cat = jnp.concatenate(
      [x_ref[...], xpl_ref[...], xpr_ref[...], xnl_ref[...], xnr_ref[...]],
      axis=1)
  h = jnp.tanh(
      jnp.dot(cat, w1_ref[...], preferred_element_type=jnp.float32)
      + b1_ref[...])
  c = (jnp.dot(h, w2_ref[...], preferred_element_type=jnp.float32)
       + b2_ref[...])
  m = jnp.max(c, axis=1, keepdims=True)
  e = jnp.exp(c - m)
  out_ref[...] = e / jnp.sum(e, axis=1, keepdims=True)


def _tc_mlp(x, xpl, xpr, xnl, xnr, W1, b1, W2, b2):
  grid = N // ROWS_BLK
  row_spec = pl.BlockSpec((ROWS_BLK, K), lambda i: (i, 0))
  half_spec = pl.BlockSpec((ROWS_BLK, KH), lambda i: (i, 0))
  full = lambda s: pl.BlockSpec(s, lambda i: (0,) * len(s))
  return pl.pallas_call(
      _mlp_body,
      grid=(grid,),
      in_specs=[
          row_spec, half_spec, half_spec, half_spec, half_spec,
          full((3 * K, 16)), full((1, 16)), full((16, K)), full((1, K)),
      ],
      out_specs=row_spec,
      out_shape=jax.ShapeDtypeStruct((N, K), jnp.float32),
  )(x, xpl, xpr, xnl, xnr, W1, b1, W2, b2)


@jax.jit
def kernel(x, pos_edge_index, neg_edge_index, W1, b1, W2, b2):
  srcp, dstp = _prep_edges(pos_edge_index)
  srcn, dstn = _prep_edges(neg_edge_index)
  xl = x[:, :KH]
  xr = x[:, KH:]
  xpl, xpr, xnl, xnr = _sc_scatter(xl, xr, srcp, dstp, srcn, dstn)
  return _tc_mlp(x, xpl, xpr, xnl, xnr, W1,
                 b1.reshape(1, 16), W2, b2.reshape(1, K))


# speedup vs baseline: 1.2993x; 1.2993x over previous
import functools

import jax
import jax.numpy as jnp
from jax import lax
from jax.experimental import pallas as pl
from jax.experimental.pallas import tpu as pltpu
from jax.experimental.pallas import tpu_sc as plsc

N = 10000
K = 128
E = 320000

NC = 2
NS = 16
KH = K // 2
CHUNK = 256
EPT = E // NS
NCHUNK = 79
EPT_PAD = NCHUNK * CHUNK
ACC_N = 10240
JUNK = N
ZROWS = ACC_N // NS
ZBLK = 128
OROWS = 624


def _prep_edges(edge_index):
  src = edge_index[0].astype(jnp.int32)
  dst = edge_index[1].astype(jnp.int32)
  pad = NS * EPT_PAD - E
  src = jnp.concatenate([src, jnp.zeros((pad,), jnp.int32)])
  dst = jnp.concatenate([dst, jnp.full((pad,), JUNK, jnp.int32)])
  return (src.reshape(NS, NCHUNK, CHUNK), dst.reshape(NS, NCHUNK, CHUNK))


def _sc_body(xl_hbm, xr_hbm, srcp, dstp, srcn, dstn,
             xpl_out, xpr_out, xnl_out, xnr_out,
             src_v, dst_v, zbuf, r0, sg0, ss0, acc):
  cid = lax.axis_index("c")
  sid = lax.axis_index("s")

  def _zrow(i, carry):
    for c in range(KH // 16):
      zbuf[i, pl.ds(c * 16, 16)] = jnp.zeros((16,), jnp.float32)
    return carry
  lax.fori_loop(0, ZBLK, _zrow, 0)

  @pl.when(cid == 0)
  def _():
    pltpu.sync_copy(srcp.at[sid], src_v)
    pltpu.sync_copy(dstp.at[sid], dst_v)

  @pl.when(cid == 1)
  def _():
    pltpu.sync_copy(srcn.at[sid], src_v)
    pltpu.sync_copy(dstn.at[sid], dst_v)

  def _pass(x_hbm, out_pos, out_neg):
    for b in range(ZROWS // ZBLK):
      pltpu.sync_copy(zbuf, acc.at[pl.ds(sid * ZROWS + b * ZBLK, ZBLK)])
    plsc.subcore_barrier()

    def _chunk(j, carry):
      pltpu.async_copy(x_hbm.at[src_v.at[j]], r0, sg0).wait()
      pltpu.async_copy(r0, acc.at[dst_v.at[j]], ss0, add=True).wait()
      return carry
    lax.fori_loop(0, NCHUNK, _chunk, 0)
    plsc.subcore_barrier()

    base = sid * OROWS
    rem_base = NS * OROWS
    rem = N - NS * OROWS

    @pl.when(cid == 0)
    def _():
      pltpu.sync_copy(acc.at[pl.ds(base, OROWS)],
                      out_pos.at[pl.ds(base, OROWS)])
      @pl.when(sid == NS - 1)
      def _():
        pltpu.sync_copy(acc.at[pl.ds(rem_base, rem)],
                        out_pos.at[pl.ds(rem_base, rem)])

    @pl.when(cid == 1)
    def _():
      pltpu.sync_copy(acc.at[pl.ds(base, OROWS)],
                      out_neg.at[pl.ds(base, OROWS)])
      @pl.when(sid == NS - 1)
      def _():
        pltpu.sync_copy(acc.at[pl.ds(rem_base, rem)],
                        out_neg.at[pl.ds(rem_base, rem)])
    plsc.subcore_barrier()

  _pass(xl_hbm, xpl_out, xnl_out)
  _pass(xr_hbm, xpr_out, xnr_out)


def _sc_scatter(xl, xr, srcp, dstp, srcn, dstn):
  mesh = plsc.VectorSubcoreMesh(core_axis_name="c", subcore_axis_name="s")
  half = jax.ShapeDtypeStruct((N, KH), jnp.float32)
  return pl.kernel(
      _sc_body,
      out_type=(half, half, half, half),
      mesh=mesh,
      compiler_params=pltpu.CompilerParams(use_tc_tiling_on_sc=False),
      scratch_types=(
          [pltpu.VMEM((NCHUNK, CHUNK), jnp.int32)] * 2
          + [pltpu.VMEM((ZBLK, KH), jnp.float32)]
          + [pltpu.VMEM((CHUNK, KH), jnp.float32)]
          + [pltpu.SemaphoreType.DMA] * 2
          + [pltpu.VMEM_SHARED((ACC_N, KH), jnp.float32)]
      ),
  )(xl, xr, srcp, dstp, srcn, dstn)


ROWS_BLK = 2000


def _mlp_body(x_ref, xpl_ref, xpr_ref, xnl_ref, xnr_ref,
              w1_ref, b1_ref, w2_ref, b2_ref, out_ref):
  cat = jnp.concatenate(
      [x_ref[...], xpl_ref[...], xpr_ref[...], xnl_ref[...], xnr_ref[...]],
      axis=1)
  h = jnp.tanh(
      jnp.dot(cat, w1_ref[...], preferred_element_type=jnp.float32)
      + b1_ref[...])
  c = (jnp.dot(h, w2_ref[...], preferred_element_type=jnp.float32)
       + b2_ref[...])
  m = jnp.max(c, axis=1, keepdims=True)
  e = jnp.exp(c - m)
  out_ref[...] = e / jnp.sum(e, axis=1, keepdims=True)


def _tc_mlp(x, xpl, xpr, xnl, xnr, W1, b1, W2, b2):
  grid = N // ROWS_BLK
  row_spec = pl.BlockSpec((ROWS_BLK, K), lambda i: (i, 0))
  half_spec = pl.BlockSpec((ROWS_BLK, KH), lambda i: (i, 0))
  full = lambda s: pl.BlockSpec(s, lambda i: (0,) * len(s))
  return pl.pallas_call(
      _mlp_body,
      grid=(grid,),
      in_specs=[
          row_spec, half_spec, half_spec, half_spec, half_spec,
          full((3 * K, 16)), full((1, 16)), full((16, K)), full((1, K)),
      ],
      out_specs=row_spec,
      out_shape=jax.ShapeDtypeStruct((N, K), jnp.float32),
  )(x, xpl, xpr, xnl, xnr, W1, b1, W2, b2)


@jax.jit
def kernel(x, pos_edge_index, neg_edge_index, W1, b1, W2, b2):
  srcp, dstp = _prep_edges(pos_edge_index)
  srcn, dstn = _prep_edges(neg_edge_index)
  xl = x[:, :KH]
  xr = x[:, KH:]
  xpl, xpr, xnl, xnr = _sc_scatter(xl, xr, srcp, dstp, srcn, dstn)
  return _tc_mlp(x, xpl, xpr, xnl, xnr, W1,
                 b1.reshape(1, 16), W2, b2.reshape(1, K))


# speedup vs baseline: 1.6445x; 1.2657x over previous
import functools

import jax
import jax.numpy as jnp
from jax import lax
from jax.experimental import pallas as pl
from jax.experimental.pallas import tpu as pltpu
from jax.experimental.pallas import tpu_sc as plsc

N = 10000
K = 128
E = 320000

NC = 2
NS = 16
CHUNK = 64
EPT = E // NS
NCHUNK = 313
EPT_PAD = NCHUNK * CHUNK
ACC_N = 10240
JUNK = N
ZROWS = ACC_N // NS
OROWS = 624


def _prep_edges(edge_index):
  src = edge_index[0].astype(jnp.int32)
  dst = edge_index[1].astype(jnp.int32)
  pad = NS * EPT_PAD - E
  src = jnp.concatenate([src, jnp.zeros((pad,), jnp.int32)])
  dst = jnp.concatenate([dst, jnp.full((pad,), JUNK, jnp.int32)])
  return (src.reshape(NS, NCHUNK, CHUNK), dst.reshape(NS, NCHUNK, CHUNK))


def _sc_body(x_hbm, srcp, dstp, srcn, dstn, xp_out, xn_out,
             src_v, dst_v, r0, sg0, ss0, acc):
  cid = lax.axis_index("c")
  sid = lax.axis_index("s")

  def _zrow(i, carry):
    for c in range(K // 16):
      r0[i, pl.ds(c * 16, 16)] = jnp.zeros((16,), jnp.float32)
    return carry
  lax.fori_loop(0, CHUNK, _zrow, 0)

  @pl.when(cid == 0)
  def _():
    pltpu.sync_copy(srcp.at[sid], src_v)
    pltpu.sync_copy(dstp.at[sid], dst_v)

  @pl.when(cid == 1)
  def _():
    pltpu.sync_copy(srcn.at[sid], src_v)
    pltpu.sync_copy(dstn.at[sid], dst_v)

  for b in range(ZROWS // CHUNK):
    pltpu.sync_copy(r0, acc.at[pl.ds(sid * ZROWS + b * CHUNK, CHUNK)])
  plsc.subcore_barrier()

  def _chunk(j, carry):
    pltpu.async_copy(x_hbm.at[src_v.at[j]], r0, sg0).wait()
    pltpu.async_copy(r0, acc.at[dst_v.at[j]], ss0, add=True).wait()
    return carry
  lax.fori_loop(0, NCHUNK, _chunk, 0)
  plsc.subcore_barrier()

  base = sid * OROWS
  rem_base = NS * OROWS
  rem = N - NS * OROWS

  @pl.when(cid == 0)
  def _():
    pltpu.sync_copy(acc.at[pl.ds(base, OROWS)],
                    xp_out.at[pl.ds(base, OROWS)])
    @pl.when(sid == NS - 1)
    def _():
      pltpu.sync_copy(acc.at[pl.ds(rem_base, rem)],
                      xp_out.at[pl.ds(rem_base, rem)])

  @pl.when(cid == 1)
  def _():
    pltpu.sync_copy(acc.at[pl.ds(base, OROWS)],
                    xn_out.at[pl.ds(base, OROWS)])
    @pl.when(sid == NS - 1)
    def _():
      pltpu.sync_copy(acc.at[pl.ds(rem_base, rem)],
                      xn_out.at[pl.ds(rem_base, rem)])


def _sc_scatter(x, srcp, dstp, srcn, dstn):
  mesh = plsc.VectorSubcoreMesh(core_axis_name="c", subcore_axis_name="s")
  full = jax.ShapeDtypeStruct((N, K), jnp.float32)
  return pl.kernel(
      _sc_body,
      out_type=(full, full),
      mesh=mesh,
      compiler_params=pltpu.CompilerParams(use_tc_tiling_on_sc=False),
      scratch_types=(
          [pltpu.VMEM((NCHUNK, CHUNK), jnp.int32)] * 2
          + [pltpu.VMEM((CHUNK, K), jnp.float32)]
          + [pltpu.SemaphoreType.DMA] * 2
          + [pltpu.VMEM_SHARED((ACC_N, K), jnp.float32)]
      ),
  )(x, srcp, dstp, srcn, dstn)


ROWS_BLK = 2000


def _mlp_body(x_ref, xp_ref, xn_ref, w1_ref, b1_ref, w2_ref, b2_ref, out_ref):
  cat = jnp.concatenate([x_ref[...], xp_ref[...], xn_ref[...]], axis=1)
  h = jnp.tanh(
      jnp.dot(cat, w1_ref[...], preferred_element_type=jnp.float32)
      + b1_ref[...])
  c = (jnp.dot(h, w2_ref[...], preferred_element_type=jnp.float32)
       + b2_ref[...])
  m = jnp.max(c, axis=1, keepdims=True)
  e = jnp.exp(c - m)
  out_ref[...] = e / jnp.sum(e, axis=1, keepdims=True)


def _tc_mlp(x, xp, xn, W1, b1, W2, b2):
  grid = N // ROWS_BLK
  row_spec = pl.BlockSpec((ROWS_BLK, K), lambda i: (i, 0))
  full = lambda s: pl.BlockSpec(s, lambda i: (0,) * len(s))
  return pl.pallas_call(
      _mlp_body,
      grid=(grid,),
      in_specs=[
          row_spec, row_spec, row_spec,
          full((3 * K, 16)), full((1, 16)), full((16, K)), full((1, K)),
      ],
      out_specs=row_spec,
      out_shape=jax.ShapeDtypeStruct((N, K), jnp.float32),
  )(x, xp, xn, W1, b1, W2, b2)


@jax.jit
def kernel(x, pos_edge_index, neg_edge_index, W1, b1, W2, b2):
  srcp, dstp = _prep_edges(pos_edge_index)
  srcn, dstn = _prep_edges(neg_edge_index)
  xp, xn = _sc_scatter(x, srcp, dstp, srcn, dstn)
  return _tc_mlp(x, xp, xn, W1,
                 b1.reshape(1, 16), W2, b2.reshape(1, K))
